# baseline (speedup 1.0000x reference)
import functools

import jax
import jax.numpy as jnp
from jax import lax
from jax.experimental import pallas as pl
from jax.experimental.pallas import tpu as pltpu
from jax.experimental.pallas import tpu_sc as plsc

B, S, D = 4, 4096, 2048
N = 128
TOTAL = B * N
L = 16
NC, NS = 2, 16
NW = NC * NS
BPW = TOTAL // NW
CHUNKS = D // L
UNROLL = 8

_mesh = plsc.VectorSubcoreMesh(core_axis_name="c", subcore_axis_name="s")


@functools.partial(
    pl.kernel,
    mesh=_mesh,
    out_type=jax.ShapeDtypeStruct((TOTAL, D), jnp.float32),
    scratch_types=[
        pltpu.VMEM((BPW,), jnp.int32),
        pltpu.VMEM((BPW,), jnp.int32),
        pltpu.VMEM((BPW, D), jnp.float32),
        pltpu.SemaphoreType.DMA,
    ],
)
def _gather_pool(wv_hbm, ids_hbm, mask_hbm, out_hbm, idx_v, mask_v, rows_v, sem):
    wid = lax.axis_index("s") * NC + lax.axis_index("c")
    base = wid * BPW
    pltpu.sync_copy(ids_hbm.at[pl.ds(base, BPW)], idx_v)
    pltpu.sync_copy(mask_hbm.at[pl.ds(base, BPW)], mask_v)
    boff = (base // N) * S
    idx_v[...] = idx_v[...] + boff
    pltpu.async_copy(wv_hbm.at[idx_v], rows_v, sem).wait()

    mask_f = mask_v[...].astype(jnp.float32)

    def row_body(r, _):
        mf = mask_f.at[jnp.full((L,), r, jnp.int32)].get(
            mode="promise_in_bounds")

        def col_body(j, _):
            for u in range(UNROLL):
                c = (j * UNROLL + u) * L
                rows_v[r, pl.ds(c, L)] = rows_v[r, pl.ds(c, L)] * mf
            return 0

        lax.fori_loop(0, CHUNKS // UNROLL, col_body, 0)
        return 0

    lax.fori_loop(0, BPW, row_body, 0)
    pltpu.sync_copy(rows_v, out_hbm.at[pl.ds(base, BPW)])


def kernel(word_vectors, sent_rep_token_ids, sent_rep_mask):
    wv2d = word_vectors.reshape(B * S, D)
    ids = sent_rep_token_ids.reshape(TOTAL)
    msk = sent_rep_mask.reshape(TOTAL)
    out = _gather_pool(wv2d, ids, msk)
    return out.reshape(B, N, D), sent_rep_mask


# speedup vs baseline: 1.1674x; 1.1674x over previous
import functools

import jax
import jax.numpy as jnp
from jax import lax
from jax.experimental import pallas as pl
from jax.experimental.pallas import tpu as pltpu
from jax.experimental.pallas import tpu_sc as plsc

B, S, D = 4, 4096, 2048
N = 128
TOTAL = B * N
L = 16
NC, NS = 2, 16
NW = NC * NS
BPW = TOTAL // NW
CHUNKS = D // L
UNROLL = 8

_mesh = plsc.VectorSubcoreMesh(core_axis_name="c", subcore_axis_name="s")


@functools.partial(
    pl.kernel,
    mesh=_mesh,
    out_type=jax.ShapeDtypeStruct((TOTAL, D), jnp.float32),
    scratch_types=[
        pltpu.VMEM((BPW,), jnp.int32),
        pltpu.VMEM((BPW,), jnp.int32),
        pltpu.VMEM((BPW, D), jnp.float32),
        pltpu.SemaphoreType.DMA,
    ],
)
def _gather_pool(wv_hbm, ids_hbm, mask_hbm, out_hbm, idx_v, mask_v, rows_v, sem):
    wid = lax.axis_index("s") * NC + lax.axis_index("c")
    base = wid * BPW
    pltpu.sync_copy(ids_hbm.at[pl.ds(base, BPW)], idx_v)
    pltpu.sync_copy(mask_hbm.at[pl.ds(base, BPW)], mask_v)
    boff = (base // N) * S
    idx_v[...] = idx_v[...] + boff
    pltpu.async_copy(wv_hbm.at[idx_v], rows_v, sem).wait()

    zero = jnp.zeros((L,), jnp.float32)
    mask_reg = mask_v[...]

    for r in range(BPW):
        @pl.when(mask_reg[r] == 0)
        def _zero_row(r=r):
            def col_body(j, _):
                for u in range(UNROLL):
                    rows_v[r, pl.ds((j * UNROLL + u) * L, L)] = zero
                return 0

            lax.fori_loop(0, CHUNKS // UNROLL, col_body, 0)
    pltpu.sync_copy(rows_v, out_hbm.at[pl.ds(base, BPW)])


def kernel(word_vectors, sent_rep_token_ids, sent_rep_mask):
    wv2d = word_vectors.reshape(B * S, D)
    ids = sent_rep_token_ids.reshape(TOTAL)
    msk = sent_rep_mask.reshape(TOTAL)
    out = _gather_pool(wv2d, ids, msk)
    return out.reshape(B, N, D), sent_rep_mask
